# baseline (speedup 1.0000x reference)
import functools

import jax
import jax.numpy as jnp
from jax.experimental import pallas as pl
from jax.experimental.pallas import tpu as pltpu

_C = 128


_P = 64


def _fourth_smallest_streamed(g, yn, xn, bm, n):
    nc = n // _C
    panels = []
    for r0 in range(0, bm, _P):
        inf = jnp.full((_P, _C), jnp.inf, dtype=jnp.float32)
        m1, m2, m3, m4 = inf, inf, inf, inf
        for c in range(nc):
            v = (yn[:, c * _C:(c + 1) * _C]
                 - 2.0 * g[r0:r0 + _P, c * _C:(c + 1) * _C])
            hi = jnp.maximum(m1, v)
            m1 = jnp.minimum(m1, v)
            hi2 = jnp.maximum(m2, hi)
            m2 = jnp.minimum(m2, hi)
            hi3 = jnp.maximum(m3, hi2)
            m3 = jnp.minimum(m3, hi2)
            m4 = jnp.minimum(m4, hi3)
        cand = jnp.concatenate([m1, m2, m3, m4], axis=1)
        m = None
        for it in range(4):
            m = jnp.min(cand, axis=1, keepdims=True)
            if it < 3:
                cand = jnp.where(cand <= m, jnp.inf, cand)
        panels.append(m)
    return jnp.concatenate(panels, axis=0) + xn


def _body(real_ref, fake_ref, rr_ref, rf_ref, met_ref,
          nr_ref, nf_ref, r2r_ref, r2f_ref, prec_ref, rec_ref,
          *, bm, nb, n):
    p = pl.program_id(0)
    i = pl.program_id(1)
    sl = pl.ds(i * bm, bm)

    @pl.when(p == 0)
    def _norms():
        rrow = real_ref[sl, :].astype(jnp.float32)
        nr_ref[0, sl] = jnp.sum(rrow * rrow, axis=1)
        frow = fake_ref[sl, :].astype(jnp.float32)
        nf_ref[0, sl] = jnp.sum(frow * frow, axis=1)

    def _gram(rows_ref, cols_ref):
        return jax.lax.dot_general(
            rows_ref[sl, :], cols_ref[...],
            dimension_numbers=(((1,), (1,)), ((), ())),
            preferred_element_type=jnp.float32)

    def _radii_phase(src_ref, norm_ref, radii_out_ref, r2_out_ref):
        g = _gram(src_ref, src_ref)
        xn = norm_ref[0, sl].reshape(bm, 1)
        v4 = _fourth_smallest_streamed(g, norm_ref[...], xn, bm, n)
        r2 = jnp.maximum(v4, 1e-12)
        r2_out_ref[0, sl] = r2[:, 0]
        radii_out_ref[0, sl] = jnp.sqrt(r2)[:, 0]

    @pl.when(p == 1)
    def _real_radii():
        _radii_phase(real_ref, nr_ref, rr_ref, r2r_ref)

    @pl.when(p == 2)
    def _fake_radii():
        _radii_phase(fake_ref, nf_ref, rf_ref, r2f_ref)

    @pl.when(p == 3)
    def _cross():
        g = _gram(fake_ref, real_ref)
        xn = nf_ref[0, sl].reshape(bm, 1)
        r2f_block = r2f_ref[0, sl].reshape(bm, 1)
        prec_acc = jnp.zeros((bm, _C), dtype=jnp.float32)
        rec_chunks = []
        for c in range(n // _C):
            d2 = (xn + nr_ref[:, c * _C:(c + 1) * _C]
                  - 2.0 * g[:, c * _C:(c + 1) * _C])
            within_real = (d2 <= r2r_ref[:, c * _C:(c + 1) * _C])
            prec_acc = jnp.maximum(prec_acc, within_real.astype(jnp.float32))
            within_fake = (d2 <= r2f_block).astype(jnp.float32)
            rec_chunks.append(jnp.max(within_fake, axis=0, keepdims=True))
        prec_ref[0, sl] = jnp.max(prec_acc, axis=1)
        rec_part = jnp.concatenate(rec_chunks, axis=1)
        rec_ref[...] = jnp.where(
            i == 0, rec_part, jnp.maximum(rec_ref[...], rec_part))

        @pl.when(i == nb - 1)
        def _():
            precision = jnp.sum(prec_ref[...]) / n
            recall = jnp.sum(rec_ref[...]) / n
            lane = jax.lax.broadcasted_iota(jnp.int32, (1, 128), 1)
            met_ref[...] = jnp.where(
                lane == 0, precision, jnp.where(lane == 1, recall, 0.0))


def kernel(real_feats, fake_feats):
    n, d = real_feats.shape
    bm = 256 if n % 256 == 0 else n
    nb = n // bm

    real_bf = real_feats.astype(jnp.bfloat16)
    fake_bf = fake_feats.astype(jnp.bfloat16)

    body = functools.partial(_body, bm=bm, nb=nb, n=n)

    full = pl.BlockSpec((n, d), lambda p, i: (0, 0))
    vec = pl.BlockSpec((1, n), lambda p, i: (0, 0))
    met = pl.BlockSpec((1, 128), lambda p, i: (0, 0))

    rr, rf, metrics = pl.pallas_call(
        body,
        grid=(4, nb),
        in_specs=[full, full],
        out_specs=[vec, vec, met],
        out_shape=[
            jax.ShapeDtypeStruct((1, n), jnp.float32),
            jax.ShapeDtypeStruct((1, n), jnp.float32),
            jax.ShapeDtypeStruct((1, 128), jnp.float32),
        ],
        scratch_shapes=[
            pltpu.VMEM((1, n), jnp.float32),
            pltpu.VMEM((1, n), jnp.float32),
            pltpu.VMEM((1, n), jnp.float32),
            pltpu.VMEM((1, n), jnp.float32),
            pltpu.VMEM((1, n), jnp.float32),
            pltpu.VMEM((1, n), jnp.float32),
        ],
        compiler_params=pltpu.CompilerParams(
            dimension_semantics=("arbitrary", "arbitrary")),
    )(real_bf, fake_bf)

    return jnp.concatenate(
        [metrics[0, :2], rr[0, :], rf[0, :]])
